# baseline (speedup 1.0000x reference)
import functools

import jax
import jax.numpy as jnp
from jax import lax
from jax.experimental import pallas as pl
from jax.experimental.pallas import tpu as pltpu
from jax.experimental.pallas import tpu_sc as plsc

_WS = 16
_N = (_WS * _WS) ** 2
_TABLE = (2 * _WS - 1) ** 2
_TABLE_PAD = 976
_NC = 1
_NS = 16
_L = 16
_NW = _NC * _NS
_B_PER_W = _N // _NW
_STEPS = _B_PER_W // _L


def _sc_gather(table_flat, idx_flat):
    mesh = plsc.VectorSubcoreMesh(
        core_axis_name="c", subcore_axis_name="s", num_cores=_NC
    )

    @functools.partial(
        pl.kernel,
        mesh=mesh,
        out_type=jax.ShapeDtypeStruct((_N,), jnp.float32),
        scratch_types=[
            pltpu.VMEM((_TABLE,), jnp.float32),
            pltpu.VMEM((_B_PER_W,), jnp.int32),
            pltpu.VMEM((_B_PER_W,), jnp.float32),
            pltpu.SemaphoreType.DMA,
            pltpu.SemaphoreType.DMA,
            pltpu.SemaphoreType.DMA,
        ],
        compiler_params=pltpu.CompilerParams(
            needs_layout_passes=False,
            disable_bounds_checks=True,
            disable_semaphore_checks=True,
        ),
    )
    def k(table_hbm, idx_hbm, out_hbm, table_v, idx_v, vals_v, sem_t, sem_i, sem_j):
        wid = lax.axis_index("s") * _NC + lax.axis_index("c")
        base = wid * _B_PER_W
        half = _B_PER_W // 2
        cp_t = pltpu.async_copy(table_hbm, table_v, sem_t)
        cp_i0 = pltpu.async_copy(
            idx_hbm.at[pl.ds(base, half)], idx_v.at[pl.ds(0, half)], sem_i
        )
        cp_i1 = pltpu.async_copy(
            idx_hbm.at[pl.ds(base + half, half)], idx_v.at[pl.ds(half, half)], sem_j
        )
        cp_t.wait()
        cp_i0.wait()

        @plsc.parallel_loop(0, _STEPS // 2, step=1, unroll=16)
        def body_lo(i):
            off = i * _L
            idx = idx_v[pl.ds(off, _L)]
            vals_v[pl.ds(off, _L)] = plsc.load_gather(table_v, [idx])

        cp_lo = pltpu.async_copy(
            vals_v.at[pl.ds(0, half)], out_hbm.at[pl.ds(base, half)], sem_t
        )
        cp_i1.wait()

        @plsc.parallel_loop(_STEPS // 2, _STEPS, step=1, unroll=16)
        def body_hi(i):
            off = i * _L
            idx = idx_v[pl.ds(off, _L)]
            vals_v[pl.ds(off, _L)] = plsc.load_gather(table_v, [idx])

        cp_hi = pltpu.async_copy(
            vals_v.at[pl.ds(half, half)], out_hbm.at[pl.ds(base + half, half)], sem_i
        )
        cp_lo.wait()
        cp_hi.wait()

    return k(table_flat, idx_flat)


def kernel(relative_position_bias_table, relative_position_index, num_heads):
    ws = _WS
    table_flat = relative_position_bias_table.reshape(-1)
    idx_flat = relative_position_index.reshape(-1)
    out = _sc_gather(table_flat, idx_flat)
    return out.reshape(1, ws * ws, ws * ws)


# speedup vs baseline: 1.0073x; 1.0073x over previous
import functools

import jax
import jax.numpy as jnp
from jax import lax
from jax.experimental import pallas as pl
from jax.experimental.pallas import tpu as pltpu
from jax.experimental.pallas import tpu_sc as plsc

_WS = 16
_N = (_WS * _WS) ** 2
_TABLE = (2 * _WS - 1) ** 2
_TABLE_PAD = 976
_NC = 1
_NS = 16
_L = 16
_NW = _NC * _NS
_B_PER_W = _N // _NW
_STEPS = _B_PER_W // _L


def _sc_gather(table_flat, idx_flat):
    mesh = plsc.VectorSubcoreMesh(
        core_axis_name="c", subcore_axis_name="s", num_cores=_NC
    )

    @functools.partial(
        pl.kernel,
        mesh=mesh,
        out_type=jax.ShapeDtypeStruct((_N,), jnp.float32),
        scratch_types=[
            pltpu.VMEM((_TABLE,), jnp.float32),
            pltpu.VMEM((_B_PER_W,), jnp.int32),
            pltpu.VMEM((_B_PER_W,), jnp.float32),
            pltpu.SemaphoreType.DMA,
            pltpu.SemaphoreType.DMA,
            pltpu.SemaphoreType.DMA,
        ],
        compiler_params=pltpu.CompilerParams(
            needs_layout_passes=False,
            disable_bounds_checks=True,
            disable_semaphore_checks=True,
        ),
    )
    def k(table_hbm, idx_hbm, out_hbm, table_v, idx_v, vals_v, sem_t, sem_i, sem_j):
        wid = lax.axis_index("s") * _NC + lax.axis_index("c")
        base = wid * _B_PER_W
        half = _B_PER_W // 2
        cp_t = pltpu.async_copy(table_hbm, table_v, sem_t)
        cp_i0 = pltpu.async_copy(
            idx_hbm.at[pl.ds(base, half)], idx_v.at[pl.ds(0, half)], sem_i
        )
        cp_i1 = pltpu.async_copy(
            idx_hbm.at[pl.ds(base + half, half)], idx_v.at[pl.ds(half, half)], sem_j
        )
        cp_t.wait()
        cp_i0.wait()

        @plsc.parallel_loop(0, _STEPS // 2, step=1, unroll=8)
        def body_lo(i):
            off = i * _L
            idx = idx_v[pl.ds(off, _L)]
            vals_v[pl.ds(off, _L)] = plsc.load_gather(table_v, [idx])

        cp_lo = pltpu.async_copy(
            vals_v.at[pl.ds(0, half)], out_hbm.at[pl.ds(base, half)], sem_t
        )
        cp_i1.wait()

        @plsc.parallel_loop(_STEPS // 2, _STEPS, step=1, unroll=8)
        def body_hi(i):
            off = i * _L
            idx = idx_v[pl.ds(off, _L)]
            vals_v[pl.ds(off, _L)] = plsc.load_gather(table_v, [idx])

        cp_hi = pltpu.async_copy(
            vals_v.at[pl.ds(half, half)], out_hbm.at[pl.ds(base + half, half)], sem_i
        )
        cp_lo.wait()
        cp_hi.wait()

    return k(table_flat, idx_flat)


def kernel(relative_position_bias_table, relative_position_index, num_heads):
    ws = _WS
    table_flat = relative_position_bias_table.reshape(-1)
    idx_flat = relative_position_index.reshape(-1)
    out = _sc_gather(table_flat, idx_flat)
    return out.reshape(1, ws * ws, ws * ws)


# speedup vs baseline: 1.0104x; 1.0031x over previous
import functools

import jax
import jax.numpy as jnp
from jax import lax
from jax.experimental import pallas as pl
from jax.experimental.pallas import tpu as pltpu
from jax.experimental.pallas import tpu_sc as plsc

_WS = 16
_N = (_WS * _WS) ** 2
_TABLE = (2 * _WS - 1) ** 2
_NC = 1
_NS = 16
_L = 16
_NW = _NC * _NS
_B_PER_W = _N // _NW
_STEPS = _B_PER_W // _L


def _sc_gather(table_flat, idx_flat):
    mesh = plsc.VectorSubcoreMesh(
        core_axis_name="c", subcore_axis_name="s", num_cores=_NC
    )

    @functools.partial(
        pl.kernel,
        mesh=mesh,
        out_type=jax.ShapeDtypeStruct((_N,), jnp.float32),
        scratch_types=[
            pltpu.VMEM((_TABLE,), jnp.float32),
            pltpu.VMEM((_B_PER_W,), jnp.int32),
            pltpu.VMEM((_B_PER_W,), jnp.float32),
            pltpu.SemaphoreType.DMA,
            pltpu.SemaphoreType.DMA,
            pltpu.SemaphoreType.DMA,
        ],
        compiler_params=pltpu.CompilerParams(
            needs_layout_passes=False,
            disable_bounds_checks=True,
            disable_semaphore_checks=True,
        ),
    )
    def k(table_hbm, idx_hbm, out_hbm, table_v, idx_v, vals_v, sem_t, sem_i, sem_j):
        wid = lax.axis_index("s") * _NC + lax.axis_index("c")
        base = wid * _B_PER_W
        half = _B_PER_W // 2
        cp_t = pltpu.async_copy(table_hbm, table_v, sem_t)
        cp_i0 = pltpu.async_copy(
            idx_hbm.at[pl.ds(base, half)], idx_v.at[pl.ds(0, half)], sem_i
        )
        cp_i1 = pltpu.async_copy(
            idx_hbm.at[pl.ds(base + half, half)], idx_v.at[pl.ds(half, half)], sem_j
        )
        cp_t.wait()
        cp_i0.wait()

        @plsc.parallel_loop(0, _STEPS // 2, step=1, unroll=8)
        def body_lo(i):
            off = i * _L
            idx = idx_v[pl.ds(off, _L)]
            vals_v[pl.ds(off, _L)] = plsc.load_gather(table_v, [idx])

        cp_lo = pltpu.async_copy(
            vals_v.at[pl.ds(0, half)], out_hbm.at[pl.ds(base, half)], sem_t
        )
        cp_i1.wait()

        @plsc.parallel_loop(_STEPS // 2, _STEPS, step=1, unroll=8)
        def body_hi(i):
            off = i * _L
            idx = idx_v[pl.ds(off, _L)]
            vals_v[pl.ds(off, _L)] = plsc.load_gather(table_v, [idx])

        cp_hi = pltpu.async_copy(
            vals_v.at[pl.ds(half, half)], out_hbm.at[pl.ds(base + half, half)], sem_i
        )
        cp_lo.wait()
        cp_hi.wait()

    return k(table_flat, idx_flat)


def kernel(relative_position_bias_table, relative_position_index, num_heads):
    ws = _WS
    table_flat = relative_position_bias_table.reshape(-1)
    idx_flat = relative_position_index.reshape(-1)
    out = _sc_gather(table_flat, idx_flat)
    return out.reshape(1, ws * ws, ws * ws)
